# Initial kernel scaffold; baseline (speedup 1.0000x reference)
import jax
import jax.numpy as jnp
from jax.experimental import pallas as pl


def kernel(x, table):
    raise NotImplementedError("write your pallas kernel here")


# speedup vs baseline: 2.9759x; 2.9759x over previous
import functools

import jax
import jax.numpy as jnp
from jax import lax
from jax.experimental import pallas as pl
from jax.experimental.pallas import tpu as pltpu
from jax.experimental.pallas import tpu_sc as plsc

_NC = 2
_NS = 16
_NW = _NC * _NS
_CHUNK = 128


@functools.lru_cache(maxsize=None)
def _make_gather(B, V, D):
    assert B % (_NW * _CHUNK) == 0
    b_per_w = B // _NW
    n_chunks = b_per_w // _CHUNK
    mesh = plsc.VectorSubcoreMesh(core_axis_name="c", subcore_axis_name="s")

    @functools.partial(
        pl.kernel,
        mesh=mesh,
        out_type=jax.ShapeDtypeStruct((B, D), jnp.float32),
        scratch_types=[
            pltpu.VMEM((b_per_w,), jnp.int32),
            pltpu.VMEM((_CHUNK, D), jnp.float32),
            pltpu.SemaphoreType.DMA,
        ],
    )
    def gather_kernel(table_hbm, idx_hbm, out_hbm, idx_v, rows_v, sem):
        wid = lax.axis_index("s") * _NC + lax.axis_index("c")
        base = wid * b_per_w
        pltpu.sync_copy(idx_hbm.at[pl.ds(base, b_per_w)], idx_v)

        def body(c, _):
            off = c * _CHUNK
            pltpu.async_copy(
                table_hbm.at[idx_v.at[pl.ds(off, _CHUNK)]], rows_v, sem
            ).wait()
            pltpu.sync_copy(rows_v, out_hbm.at[pl.ds(base + off, _CHUNK)])
            return ()

        lax.fori_loop(0, n_chunks, body, (), unroll=False)

    return gather_kernel


def kernel(x, table):
    bs, seq = x.shape
    V, D = table.shape
    idx = x.reshape(bs * seq).astype(jnp.int32)
    out = _make_gather(bs * seq, V, D)(table, idx)
    return out.reshape(bs, seq, D)


# speedup vs baseline: 3.3106x; 1.1125x over previous
import functools

import jax
import jax.numpy as jnp
from jax import lax
from jax.experimental import pallas as pl
from jax.experimental.pallas import tpu as pltpu
from jax.experimental.pallas import tpu_sc as plsc

_NC = 2
_NS = 16
_NW = _NC * _NS
_CHUNK = 128


_NBUF = 5


@functools.lru_cache(maxsize=None)
def _make_gather(B, V, D):
    assert B % (_NW * _CHUNK * _NBUF) == 0
    b_per_w = B // _NW
    n_rounds = b_per_w // (_CHUNK * _NBUF)
    mesh = plsc.VectorSubcoreMesh(core_axis_name="c", subcore_axis_name="s")

    @functools.partial(
        pl.kernel,
        mesh=mesh,
        out_type=jax.ShapeDtypeStruct((B, D), jnp.float32),
        scratch_types=[
            pltpu.VMEM((b_per_w,), jnp.int32),
            pltpu.VMEM((_NBUF, _CHUNK, D), jnp.float32),
            [pltpu.SemaphoreType.DMA] * _NBUF,
            [pltpu.SemaphoreType.DMA] * _NBUF,
        ],
    )
    def gather_kernel(table_hbm, idx_hbm, out_hbm, idx_v, rows_v, gsems, wsems):
        wid = lax.axis_index("s") * _NC + lax.axis_index("c")
        base = wid * b_per_w
        pltpu.sync_copy(idx_hbm.at[pl.ds(base, b_per_w)], idx_v)

        def issue_gather(c, j):
            pltpu.async_copy(
                table_hbm.at[idx_v.at[pl.ds(c * _CHUNK, _CHUNK)]],
                rows_v.at[j],
                gsems[j],
            )

        def wait_gather(j):
            pltpu.make_async_copy(
                table_hbm.at[idx_v.at[pl.ds(0, _CHUNK)]], rows_v.at[j], gsems[j]
            ).wait()

        def issue_write(c, j):
            pltpu.async_copy(
                rows_v.at[j], out_hbm.at[pl.ds(base + c * _CHUNK, _CHUNK)], wsems[j]
            )

        def wait_write(j):
            pltpu.make_async_copy(
                rows_v.at[j], out_hbm.at[pl.ds(base, _CHUNK)], wsems[j]
            ).wait()

        for j in range(_NBUF):
            issue_gather(j, j)

        def round_body(r, _):
            for j in range(_NBUF):
                wait_gather(j)
                issue_write(r * _NBUF + j, j)
            for j in range(_NBUF):
                wait_write(j)
                issue_gather((r + 1) * _NBUF + j, j)
            return ()

        lax.fori_loop(0, n_rounds - 1, round_body, (), unroll=False)

        last = (n_rounds - 1) * _NBUF
        for j in range(_NBUF):
            wait_gather(j)
            issue_write(last + j, j)
        for j in range(_NBUF):
            wait_write(j)

    return gather_kernel


def kernel(x, table):
    bs, seq = x.shape
    V, D = table.shape
    idx = x.reshape(bs * seq).astype(jnp.int32)
    out = _make_gather(bs * seq, V, D)(table, idx)
    return out.reshape(bs, seq, D)


# speedup vs baseline: 5.8854x; 1.7777x over previous
import functools

import jax
import jax.numpy as jnp
from jax import lax
from jax.experimental import pallas as pl
from jax.experimental.pallas import tpu as pltpu
from jax.experimental.pallas import tpu_sc as plsc

_NC = 2
_NS = 16
_NW = _NC * _NS
_NB = 2
_NBUF = 4


@functools.lru_cache(maxsize=None)
def _make_gather(BS, T, V, D):
    assert BS % (_NW * _NB * _NBUF) == 0
    b_per_w = BS // _NW
    n_rounds = b_per_w // (_NB * _NBUF)
    mesh = plsc.VectorSubcoreMesh(core_axis_name="c", subcore_axis_name="s")

    @functools.partial(
        pl.kernel,
        mesh=mesh,
        out_type=jax.ShapeDtypeStruct((BS, T, D), jnp.float32),
        scratch_types=[
            pltpu.VMEM((b_per_w, T), jnp.int32),
            pltpu.VMEM((_NBUF, _NB, T, D), jnp.float32),
            [pltpu.SemaphoreType.DMA] * _NBUF,
            [pltpu.SemaphoreType.DMA] * _NBUF,
        ],
    )
    def gather_kernel(table_hbm, idx_hbm, out_hbm, idx_v, rows_v, gsems, wsems):
        wid = lax.axis_index("s") * _NC + lax.axis_index("c")
        base_b = wid * b_per_w
        pltpu.sync_copy(idx_hbm.at[pl.ds(base_b, b_per_w)], idx_v)

        def issue_gathers(chunk, j):
            for jb in range(_NB):
                pltpu.async_copy(
                    table_hbm.at[idx_v.at[chunk * _NB + jb]],
                    rows_v.at[j, jb],
                    gsems[j],
                )

        def wait_gathers(j):
            for jb in range(_NB):
                pltpu.make_async_copy(
                    table_hbm.at[idx_v.at[0]], rows_v.at[j, jb], gsems[j]
                ).wait()

        def issue_write(chunk, j):
            pltpu.async_copy(
                rows_v.at[j],
                out_hbm.at[pl.ds(base_b + chunk * _NB, _NB)],
                wsems[j],
            )

        def wait_write(j):
            pltpu.make_async_copy(
                rows_v.at[j], out_hbm.at[pl.ds(base_b, _NB)], wsems[j]
            ).wait()

        for j in range(_NBUF):
            issue_gathers(j, j)

        def round_body(r, _):
            for j in range(_NBUF):
                wait_gathers(j)
                issue_write(r * _NBUF + j, j)
            for j in range(_NBUF):
                wait_write(j)
                issue_gathers((r + 1) * _NBUF + j, j)
            return ()

        lax.fori_loop(0, n_rounds - 1, round_body, (), unroll=False)

        last = (n_rounds - 1) * _NBUF
        for j in range(_NBUF):
            wait_gathers(j)
            issue_write(last + j, j)
        for j in range(_NBUF):
            wait_write(j)

    return gather_kernel


def kernel(x, table):
    bs, seq = x.shape
    V, D = table.shape
    return _make_gather(bs, seq, V, D)(table, x.astype(jnp.int32))


# speedup vs baseline: 10.3738x; 1.7626x over previous
import functools

import jax
import jax.numpy as jnp
from jax import lax
from jax.experimental import pallas as pl
from jax.experimental.pallas import tpu as pltpu
from jax.experimental.pallas import tpu_sc as plsc

_NC = 2
_NS = 16
_NW = _NC * _NS
_NBUF = 5


@functools.lru_cache(maxsize=None)
def _make_gather(BS, T, V, D):
    assert BS % _NW == 0 and T % _NBUF == 0
    b_per_w = BS // _NW
    n_rounds = T // _NBUF
    mesh = plsc.VectorSubcoreMesh(core_axis_name="c", subcore_axis_name="s")

    @functools.partial(
        pl.kernel,
        mesh=mesh,
        out_type=jax.ShapeDtypeStruct((T, BS, D), jnp.float32),
        scratch_types=[
            pltpu.VMEM((T, b_per_w), jnp.int32),
            pltpu.VMEM((_NBUF, b_per_w, D), jnp.float32),
            [pltpu.SemaphoreType.DMA] * _NBUF,
            [pltpu.SemaphoreType.DMA] * _NBUF,
        ],
    )
    def gather_kernel(table_hbm, idx_hbm, out_hbm, idx_v, rows_v, gsems, wsems):
        wid = lax.axis_index("s") * _NC + lax.axis_index("c")
        base_b = wid * b_per_w
        pltpu.sync_copy(idx_hbm.at[:, pl.ds(base_b, b_per_w)], idx_v)

        def issue_gather(t, j):
            pltpu.async_copy(
                table_hbm.at[idx_v.at[t]], rows_v.at[j], gsems[j]
            )

        def wait_gather(j):
            pltpu.make_async_copy(
                table_hbm.at[idx_v.at[0]], rows_v.at[j], gsems[j]
            ).wait()

        def issue_write(t, j):
            pltpu.async_copy(
                rows_v.at[j], out_hbm.at[t, pl.ds(base_b, b_per_w)], wsems[j]
            )

        def wait_write(j):
            pltpu.make_async_copy(
                rows_v.at[j], out_hbm.at[0, pl.ds(base_b, b_per_w)], wsems[j]
            ).wait()

        for j in range(_NBUF):
            issue_gather(j, j)

        def round_body(r, _):
            for j in range(_NBUF):
                wait_gather(j)
                issue_write(r * _NBUF + j, j)
            for j in range(_NBUF):
                wait_write(j)
                issue_gather((r + 1) * _NBUF + j, j)
            return ()

        lax.fori_loop(0, n_rounds - 1, round_body, (), unroll=False)

        last = (n_rounds - 1) * _NBUF
        for j in range(_NBUF):
            wait_gather(j)
            issue_write(last + j, j)
        for j in range(_NBUF):
            wait_write(j)

    return gather_kernel


def kernel(x, table):
    bs, seq = x.shape
    V, D = table.shape
    xt = x.astype(jnp.int32).T
    out = _make_gather(bs, seq, V, D)(table, xt)
    return out.transpose(1, 0, 2)


# speedup vs baseline: 10.6828x; 1.0298x over previous
import functools

import jax
import jax.numpy as jnp
from jax import lax
from jax.experimental import pallas as pl
from jax.experimental.pallas import tpu as pltpu
from jax.experimental.pallas import tpu_sc as plsc

_NC = 2
_NS = 16
_NW = _NC * _NS
_NBUF = 10
_LA = 5
_SPLIT = 2


@functools.lru_cache(maxsize=None)
def _make_gather(BS, T, V, D):
    assert BS % (_NW * _SPLIT) == 0
    b_chunk = BS // (_NW * _SPLIT)
    n_chunks = T * _SPLIT
    assert n_chunks % _NBUF == 0
    n_rounds = n_chunks // _NBUF
    mesh = plsc.VectorSubcoreMesh(core_axis_name="c", subcore_axis_name="s")

    @functools.partial(
        pl.kernel,
        mesh=mesh,
        out_type=jax.ShapeDtypeStruct((T, BS, D), jnp.float32),
        scratch_types=[
            pltpu.VMEM((T, b_chunk * _SPLIT), jnp.int32),
            pltpu.VMEM((_NBUF, b_chunk, D), jnp.float32),
            [pltpu.SemaphoreType.DMA] * _NBUF,
            [pltpu.SemaphoreType.DMA] * _NBUF,
        ],
    )
    def gather_kernel(table_hbm, idx_hbm, out_hbm, idx_v, rows_v, gsems, wsems):
        wid = lax.axis_index("s") * _NC + lax.axis_index("c")
        base_b = wid * b_chunk * _SPLIT
        pltpu.sync_copy(idx_hbm.at[:, pl.ds(base_b, b_chunk * _SPLIT)], idx_v)

        def issue_gather(c, j, h):
            pltpu.async_copy(
                table_hbm.at[idx_v.at[c // _SPLIT, pl.ds(h * b_chunk, b_chunk)]],
                rows_v.at[j],
                gsems[j],
            )

        def wait_gather(j):
            pltpu.make_async_copy(
                table_hbm.at[idx_v.at[0, pl.ds(0, b_chunk)]], rows_v.at[j], gsems[j]
            ).wait()

        def issue_write(c, j, h):
            pltpu.async_copy(
                rows_v.at[j],
                out_hbm.at[c // _SPLIT, pl.ds(base_b + h * b_chunk, b_chunk)],
                wsems[j],
            )

        def wait_write(j):
            pltpu.make_async_copy(
                rows_v.at[j], out_hbm.at[0, pl.ds(base_b, b_chunk)], wsems[j]
            ).wait()

        for j in range(_LA):
            issue_gather(j, j, j % _SPLIT)

        def consume_prefetch(r, j, first_round):
            c = r * _NBUF + j
            wait_gather(j)
            issue_write(c, j, j % _SPLIT)
            cp = c + _LA
            jp = (j + _LA) % _NBUF
            if not (first_round and j < _NBUF - _LA):
                wait_write(jp)
            issue_gather(cp, jp, (j + _LA) % _SPLIT)

        for j in range(_NBUF):
            consume_prefetch(0, j, True)

        def round_body(r, _):
            for j in range(_NBUF):
                consume_prefetch(r, j, False)
            return ()

        lax.fori_loop(1, n_rounds - 1, round_body, (), unroll=False)

        last = (n_rounds - 1) * _NBUF
        for j in range(_NBUF):
            c = last + j
            wait_gather(j)
            issue_write(c, j, j % _SPLIT)
            if j < _NBUF - _LA:
                jp = (j + _LA) % _NBUF
                wait_write(jp)
                issue_gather(c + _LA, jp, (j + _LA) % _SPLIT)
        for j in range(_NBUF):
            wait_write(j)

    return gather_kernel


def kernel(x, table):
    bs, seq = x.shape
    V, D = table.shape
    xt = x.astype(jnp.int32).T
    out = _make_gather(bs, seq, V, D)(table, xt)
    return out.transpose(1, 0, 2)


# speedup vs baseline: 10.7315x; 1.0046x over previous
import functools

import jax
import jax.numpy as jnp
from jax import lax
from jax.experimental import pallas as pl
from jax.experimental.pallas import tpu as pltpu
from jax.experimental.pallas import tpu_sc as plsc

_NC = 2
_NS = 16
_NW = _NC * _NS
_NBUF = 10
_LA = 7
_SPLIT = 2


@functools.lru_cache(maxsize=None)
def _make_gather(BS, T, V, D):
    assert BS % (_NW * _SPLIT) == 0
    b_chunk = BS // (_NW * _SPLIT)
    n_chunks = T * _SPLIT
    assert n_chunks % _NBUF == 0
    n_rounds = n_chunks // _NBUF
    mesh = plsc.VectorSubcoreMesh(core_axis_name="c", subcore_axis_name="s")

    @functools.partial(
        pl.kernel,
        mesh=mesh,
        out_type=jax.ShapeDtypeStruct((T, BS, D), jnp.float32),
        scratch_types=[
            pltpu.VMEM((T, b_chunk * _SPLIT), jnp.int32),
            pltpu.VMEM((_NBUF, b_chunk, D), jnp.float32),
            [pltpu.SemaphoreType.DMA] * _NBUF,
            [pltpu.SemaphoreType.DMA] * _NBUF,
        ],
    )
    def gather_kernel(table_hbm, idx_hbm, out_hbm, idx_v, rows_v, gsems, wsems):
        wid = lax.axis_index("s") * _NC + lax.axis_index("c")
        base_b = wid * b_chunk * _SPLIT
        pltpu.sync_copy(idx_hbm.at[:, pl.ds(base_b, b_chunk * _SPLIT)], idx_v)

        def issue_gather(c, j, h):
            pltpu.async_copy(
                table_hbm.at[idx_v.at[c // _SPLIT, pl.ds(h * b_chunk, b_chunk)]],
                rows_v.at[j],
                gsems[j],
            )

        def wait_gather(j):
            pltpu.make_async_copy(
                table_hbm.at[idx_v.at[0, pl.ds(0, b_chunk)]], rows_v.at[j], gsems[j]
            ).wait()

        def issue_write(c, j, h):
            pltpu.async_copy(
                rows_v.at[j],
                out_hbm.at[c // _SPLIT, pl.ds(base_b + h * b_chunk, b_chunk)],
                wsems[j],
            )

        def wait_write(j):
            pltpu.make_async_copy(
                rows_v.at[j], out_hbm.at[0, pl.ds(base_b, b_chunk)], wsems[j]
            ).wait()

        for j in range(_LA):
            issue_gather(j, j, j % _SPLIT)

        def consume_prefetch(r, j, first_round):
            c = r * _NBUF + j
            wait_gather(j)
            issue_write(c, j, j % _SPLIT)
            cp = c + _LA
            jp = (j + _LA) % _NBUF
            if not (first_round and j < _NBUF - _LA):
                wait_write(jp)
            issue_gather(cp, jp, (j + _LA) % _SPLIT)

        for j in range(_NBUF):
            consume_prefetch(0, j, True)

        def round_body(r, _):
            for j in range(_NBUF):
                consume_prefetch(r, j, False)
            return ()

        lax.fori_loop(1, n_rounds - 1, round_body, (), unroll=False)

        last = (n_rounds - 1) * _NBUF
        for j in range(_NBUF):
            c = last + j
            wait_gather(j)
            issue_write(c, j, j % _SPLIT)
            if j < _NBUF - _LA:
                jp = (j + _LA) % _NBUF
                wait_write(jp)
                issue_gather(c + _LA, jp, (j + _LA) % _SPLIT)
        for j in range(_NBUF):
            wait_write(j)

    return gather_kernel


def kernel(x, table):
    bs, seq = x.shape
    V, D = table.shape
    xt = x.astype(jnp.int32).T
    out = _make_gather(bs, seq, V, D)(table, xt)
    return out.transpose(1, 0, 2)
